# Initial kernel scaffold; baseline (speedup 1.0000x reference)
import jax
import jax.numpy as jnp
from jax.experimental import pallas as pl


def kernel(input, target):
    raise NotImplementedError("write your pallas kernel here")


# speedup vs baseline: 1.1507x; 1.1507x over previous
import functools

import jax
import jax.numpy as jnp
from jax import lax
from jax.experimental import pallas as pl
from jax.experimental.pallas import tpu as pltpu
from jax.experimental.pallas import tpu_sc as plsc

C = 100
N = 160000
D = 256
L = 16
NC = 2
NS = 16
NW = NC * NS
RPT = 8
CP = RPT * NS
NG = N // L
G_LO = NG // NW
G_HI = G_LO + 1
NHI = NG - G_LO * NW
CHUNK = 40
NCH = (N // NW) // CHUNK

_mesh = plsc.VectorSubcoreMesh(core_axis_name="c", subcore_axis_name="s")


@functools.partial(
    pl.kernel,
    out_type=[
        jax.ShapeDtypeStruct((NW, CP, D), jnp.float32),
        jax.ShapeDtypeStruct((NW, CP, L), jnp.float32),
    ],
    mesh=_mesh,
    scratch_types=[
        pltpu.VMEM((L,), jnp.int32),
        pltpu.VMEM((L, D), jnp.float32),
        pltpu.VMEM((CP, D), jnp.float32),
        pltpu.VMEM((CP, L), jnp.float32),
    ],
)
def _phase1(x_hbm, tgt_hbm, sums_hbm, cnts_hbm, idx_v, rowbuf, table, ctable):
    cid = lax.axis_index("c")
    sid = lax.axis_index("s")
    wid = cid * NS + sid

    zero = jnp.zeros((L,), jnp.float32)
    one = jnp.ones((L,), jnp.float32)

    @pl.loop(0, CP)
    def _(c):
        ctable[c, :] = zero
        for j in range(D // L):
            table[c, pl.ds(j * L, L)] = zero

    ngroups = jnp.where(wid < NHI, G_HI, G_LO)
    base = jnp.where(wid < NHI, wid * G_HI, NHI * G_HI + (wid - NHI) * G_LO) * L

    @pl.loop(0, ngroups)
    def _(g):
        off = base + g * L
        pltpu.sync_copy(tgt_hbm.at[pl.ds(off, L)], idx_v)
        pltpu.sync_copy(x_hbm.at[pl.ds(off, L)], rowbuf)
        tvec = idx_v[...]
        for l in range(L):
            t = tvec[l]
            ctable[t, :] = ctable[t, :] + one
            for j in range(D // L):
                s = pl.ds(j * L, L)
                table[t, s] = table[t, s] + rowbuf[l, s]

    pltpu.sync_copy(table, sums_hbm.at[wid])
    pltpu.sync_copy(ctable, cnts_hbm.at[wid])


@functools.partial(
    pl.kernel,
    out_type=[
        jax.ShapeDtypeStruct((N, D), jnp.float32),
        jax.ShapeDtypeStruct((NC, CP, D), jnp.float32),
    ],
    mesh=_mesh,
    scratch_types=[
        pltpu.VMEM((RPT, D), jnp.float32),
        pltpu.VMEM((RPT, D), jnp.float32),
        pltpu.VMEM((RPT, L), jnp.float32),
        pltpu.VMEM((RPT, L), jnp.float32),
        pltpu.VMEM((CHUNK,), jnp.int32),
        pltpu.VMEM((CHUNK, D), jnp.float32),
    ],
)
def _phase2(sums_hbm, cnts_hbm, tgt_hbm, out_hbm, means_hbm,
            a_buf, b_buf, ca, cb, idx_v, rowbuf):
    cid = lax.axis_index("c")
    sid = lax.axis_index("s")
    wid = cid * NS + sid

    pltpu.sync_copy(sums_hbm.at[0, pl.ds(sid * RPT, RPT)], a_buf)
    pltpu.sync_copy(cnts_hbm.at[0, pl.ds(sid * RPT, RPT)], ca)

    @pl.loop(1, NW)
    def _(w):
        pltpu.sync_copy(sums_hbm.at[w, pl.ds(sid * RPT, RPT)], b_buf)
        pltpu.sync_copy(cnts_hbm.at[w, pl.ds(sid * RPT, RPT)], cb)
        for r in range(RPT):
            ca[r, :] = ca[r, :] + cb[r, :]
            for j in range(D // L):
                s = pl.ds(j * L, L)
                a_buf[r, s] = a_buf[r, s] + b_buf[r, s]

    for r in range(RPT):
        inv = 1.0 / jnp.maximum(ca[r, :], 1.0)
        for j in range(D // L):
            s = pl.ds(j * L, L)
            a_buf[r, s] = a_buf[r, s] * inv
    pltpu.sync_copy(a_buf, means_hbm.at[cid, pl.ds(sid * RPT, RPT)])
    plsc.subcore_barrier()

    base = wid * (N // NW)

    @pl.loop(0, NCH)
    def _(j):
        off = base + j * CHUNK
        pltpu.sync_copy(tgt_hbm.at[pl.ds(off, CHUNK)], idx_v)
        pltpu.sync_copy(means_hbm.at[cid].at[idx_v], rowbuf)
        pltpu.sync_copy(rowbuf, out_hbm.at[pl.ds(off, CHUNK)])


def kernel(input, target):
    sums, cnts = _phase1(input, target)
    out, _ = _phase2(sums, cnts, target)
    return out
